# baseline (speedup 1.0000x reference)
import functools

import jax
import jax.numpy as jnp
from jax import lax
from jax.experimental import pallas as pl
from jax.experimental.pallas import tpu as pltpu
from jax.experimental.pallas import tpu_sc as plsc

B, V, C, S = 4096, 100, 64, 1000
NC, NS, L = 2, 16, 16
NW = NC * NS
PER = B // NW
NBUF = 8


@functools.partial(
    pl.kernel,
    out_type=jax.ShapeDtypeStruct((B, V, 1, C), jnp.float32),
    mesh=plsc.VectorSubcoreMesh(core_axis_name="c", subcore_axis_name="s"),
    scratch_types=[
        pltpu.VMEM((PER, V), jnp.int32),
        pltpu.VMEM((PER, V), jnp.int32),
        pltpu.VMEM((NBUF, V, 1, C), jnp.float32),
        pltpu.SemaphoreType.DMA((NBUF,)),
        pltpu.SemaphoreType.DMA,
    ],
    compiler_params=pltpu.CompilerParams(use_tc_tiling_on_sc=False),
)
def _sc_gather(x_hbm, tab_hbm, out_hbm, xbuf, idx_v, rows_v, gsem, ssem):
    wid = lax.axis_index("s") * NC + lax.axis_index("c")
    base = wid * PER
    pltpu.sync_copy(x_hbm.at[pl.ds(base, PER)], xbuf)
    lane_off = lax.iota(jnp.int32, L) * S

    starts = [0, 16, 32, 48, 64, 80, V - L]

    def to_indices(r):
        for c0 in starts:
            idx_v[r, pl.ds(c0, L)] = xbuf[r, pl.ds(c0, L)] + (c0 * S + lane_off)

    def fire_gather(j, b):
        pltpu.async_copy(tab_hbm.at[idx_v.at[j]], rows_v.at[b], gsem.at[b])

    def wait_gather(j, b):
        pltpu.make_async_copy(
            tab_hbm.at[idx_v.at[j]], rows_v.at[b], gsem.at[b]).wait()

    def store(j, b):
        pltpu.async_copy(rows_v.at[b], out_hbm.at[base + j], ssem).wait()

    for b in range(NBUF):
        to_indices(b)
        fire_gather(b, b)

    def transform_rest(r, carry):
        to_indices(r)
        return carry

    lax.fori_loop(NBUF, PER, transform_rest, 0)

    def round_fn(gi, carry):
        g = gi * NBUF
        for b in range(NBUF):
            j = g + b
            wait_gather(j, b)
            store(j, b)
            fire_gather(j + NBUF, b)
        return carry

    lax.fori_loop(0, (PER - NBUF) // NBUF, round_fn, 0)

    for b in range(NBUF):
        j = PER - NBUF + b
        wait_gather(j, b)
        store(j, b)


def kernel(x, weight):
    tab = jnp.transpose(weight.reshape(V, C, S), (0, 2, 1)).reshape(V * S, 1, C)
    return _sc_gather(x, tab)


# speedup vs baseline: 2.8392x; 2.8392x over previous
import functools

import jax
import jax.numpy as jnp
from jax import lax
from jax.experimental import pallas as pl
from jax.experimental.pallas import tpu as pltpu
from jax.experimental.pallas import tpu_sc as plsc

B, V, C, S = 4096, 100, 64, 1000
VS = V * S
NC, NS, L = 2, 16, 16
NW = NC * NS
PER = B // NW
VP = 128
NBUF = 8


@functools.partial(
    pl.kernel,
    out_type=jax.ShapeDtypeStruct((B, V, VP), jnp.float32),
    mesh=plsc.VectorSubcoreMesh(core_axis_name="c", subcore_axis_name="s"),
    scratch_types=[
        pltpu.VMEM((PER, VP), jnp.int32),
        pltpu.VMEM((NBUF, V, VP), jnp.float32),
        pltpu.SemaphoreType.DMA((NBUF,)),
        pltpu.SemaphoreType.DMA,
    ],
    compiler_params=pltpu.CompilerParams(use_tc_tiling_on_sc=True),
)
def _sc_gather(x_hbm, tab_hbm, out_hbm, idx_v, rows_v, gsem, ssem):
    wid = lax.axis_index("s") * NC + lax.axis_index("c")
    base = wid * PER
    pltpu.sync_copy(x_hbm.at[pl.ds(base, PER)], idx_v)
    lane_off = lax.iota(jnp.int32, L) * S

    def to_indices(r):
        for c0 in range(0, VP, L):
            raw = idx_v[r, pl.ds(c0, L)] + (c0 * S + lane_off)
            idx_v[r, pl.ds(c0, L)] = jnp.minimum(raw, VS - 1)

    def fire_gather(j, b):
        pltpu.async_copy(
            tab_hbm.at[idx_v.at[j, pl.ds(0, V)]], rows_v.at[b], gsem.at[b])

    def wait_gather(j, b):
        pltpu.make_async_copy(
            tab_hbm.at[idx_v.at[j, pl.ds(0, V)]], rows_v.at[b],
            gsem.at[b]).wait()

    def store(j, b):
        pltpu.async_copy(rows_v.at[b], out_hbm.at[base + j], ssem).wait()

    for b in range(NBUF):
        to_indices(b)
        fire_gather(b, b)

    def transform_rest(r, carry):
        to_indices(r)
        return carry

    lax.fori_loop(NBUF, PER, transform_rest, 0)

    def round_fn(gi, carry):
        g = gi * NBUF
        for b in range(NBUF):
            j = g + b
            wait_gather(j, b)
            store(j, b)
            fire_gather(j + NBUF, b)
        return carry

    lax.fori_loop(0, (PER - NBUF) // NBUF, round_fn, 0)

    for b in range(NBUF):
        j = PER - NBUF + b
        wait_gather(j, b)
        store(j, b)


def kernel(x, weight):
    tabT = jnp.transpose(weight.reshape(V, C, S), (0, 2, 1)).reshape(VS, C)
    tab = jnp.pad(tabT, ((0, 0), (0, VP - C)))
    xp = jnp.pad(x, ((0, 0), (0, VP - V)))
    out = _sc_gather(xp, tab)
    return out[:, :, :C].reshape(B, V, 1, C)
